# baseline (speedup 1.0000x reference)
import functools

import jax
import jax.numpy as jnp
from jax import lax
from jax.experimental import pallas as pl
from jax.experimental.pallas import tpu as pltpu
from jax.experimental.pallas import tpu_sc as plsc

_MOVE_PROB = 0.12
_NC, _NS = 2, 16
_NW = _NC * _NS


def _make_sc_gather(BT, C):
    RW = BT // _NW
    CH = 16
    NBUF = 8
    AHD = 4
    NCH = RW // CH

    mesh = plsc.VectorSubcoreMesh(core_axis_name="c", subcore_axis_name="s")

    @functools.partial(
        pl.kernel,
        mesh=mesh,
        out_type=jax.ShapeDtypeStruct((BT, C), jnp.float32),
        scratch_types=(
            [pltpu.VMEM((RW,), jnp.int32)]
            + [pltpu.VMEM((CH, C), jnp.float32)] * NBUF
            + [pltpu.SemaphoreType.DMA] * NBUF
            + [pltpu.SemaphoreType.DMA] * NBUF
        ),
    )
    def k(x_hbm, idx_hbm, out_hbm, idx_v, *bufsem):
        bufs = bufsem[:NBUF]
        gsem = bufsem[NBUF:2 * NBUF]
        ssem = bufsem[2 * NBUF:]
        wid = lax.axis_index("s") * _NC + lax.axis_index("c")
        base = wid * RW
        pltpu.sync_copy(idx_hbm.at[pl.ds(base, RW)], idx_v)

        def start_g(c, b):
            pltpu.async_copy(
                x_hbm.at[idx_v.at[pl.ds(c * CH, CH)]], bufs[b], gsem[b])

        def wait_g(c, b):
            pltpu.make_async_copy(
                x_hbm.at[idx_v.at[pl.ds(c * CH, CH)]], bufs[b], gsem[b]).wait()

        def start_s(c, b):
            pltpu.async_copy(
                bufs[b], out_hbm.at[pl.ds(base + c * CH, CH)], ssem[b])

        def wait_s(c, b):
            pltpu.make_async_copy(
                bufs[b], out_hbm.at[pl.ds(base + c * CH, CH)], ssem[b]).wait()

        for c in range(AHD):
            start_g(c, c)

        def step(c, b, first, last):
            wait_g(c, b)
            start_s(c, b)
            if not last:
                bn = (b + AHD) % NBUF
                if not first:
                    wait_s(c + AHD - NBUF, bn)
                start_g(c + AHD, bn)

        for c in range(NBUF - AHD):
            step(c, c, True, False)

        def body(jj, carry):
            c0 = NBUF * jj + (NBUF - AHD)
            for b in range(NBUF):
                step(c0 + b, (NBUF - AHD + b) % NBUF, False, False)
            return carry

        lax.fori_loop(0, (NCH - NBUF) // NBUF, body, 0)
        for c in range(NCH - AHD, NCH):
            step(c, c % NBUF, False, True)
        for c in range(NCH - NBUF, NCH):
            wait_s(c, c % NBUF)

    return k


def kernel(x, training):
    B, T, C = x.shape
    BT = B * T
    xf = x.reshape(BT, C)

    logp = jnp.log(jnp.array(
        [_MOVE_PROB / 2.0, 1.0 - _MOVE_PROB, _MOVE_PROB / 2.0],
        dtype=jnp.float32))
    step = jax.random.categorical(
        jax.random.key(42), logp, shape=(BT,)).astype(jnp.int32) - 1
    iota = jnp.arange(BT, dtype=jnp.int32)
    idx = iota + step
    idx = idx + 2 * (idx < 0).astype(jnp.int32)
    idx = idx - 2 * (idx >= BT).astype(jnp.int32)
    idx = jnp.where(training != 0, idx, iota)
    out = _make_sc_gather(BT, C)(xf, idx)
    return out.reshape(B, T, C)


# speedup vs baseline: 1.0577x; 1.0577x over previous
import functools

import jax
import jax.numpy as jnp
from jax import lax
from jax.experimental import pallas as pl
from jax.experimental.pallas import tpu as pltpu
from jax.experimental.pallas import tpu_sc as plsc

_MOVE_PROB = 0.12
_NC, _NS = 2, 16
_NW = _NC * _NS


def _make_sc_gather(BT, C):
    RW = BT // _NW
    CH = 16
    NBUF = 8
    AHD = 4
    NCH = RW // CH

    mesh = plsc.VectorSubcoreMesh(core_axis_name="c", subcore_axis_name="s")

    @functools.partial(
        pl.kernel,
        mesh=mesh,
        out_type=jax.ShapeDtypeStruct((BT, C), jnp.float32),
        scratch_types=(
            [pltpu.VMEM((RW,), jnp.int32)]
            + [pltpu.VMEM((CH, C), jnp.float32)] * NBUF
            + [pltpu.SemaphoreType.DMA] * NBUF
            + [pltpu.SemaphoreType.DMA] * NBUF
        ),
    )
    def k(x_hbm, idx_hbm, out_hbm, idx_v, *bufsem):
        bufs = bufsem[:NBUF]
        gsem = bufsem[NBUF:2 * NBUF]
        ssem = bufsem[2 * NBUF:]
        wid = lax.axis_index("s") * _NC + lax.axis_index("c")
        base = wid * RW
        pltpu.sync_copy(idx_hbm.at[pl.ds(base, RW)], idx_v)

        def start_g(c, b):
            pltpu.async_copy(
                x_hbm.at[idx_v.at[pl.ds(c * CH, CH)]], bufs[b], gsem[b])

        def wait_g(c, b):
            pltpu.make_async_copy(
                x_hbm.at[idx_v.at[pl.ds(c * CH, CH)]], bufs[b], gsem[b]).wait()

        def start_s(c, b):
            pltpu.async_copy(
                bufs[b], out_hbm.at[pl.ds(base + c * CH, CH)], ssem[b])

        def wait_s(c, b):
            pltpu.make_async_copy(
                bufs[b], out_hbm.at[pl.ds(base + c * CH, CH)], ssem[b]).wait()

        for c in range(AHD):
            start_g(c, c)

        def step(c, b, first, last):
            wait_g(c, b)
            start_s(c, b)
            if not last:
                bn = (b + AHD) % NBUF
                if not first:
                    wait_s(c + AHD - NBUF, bn)
                start_g(c + AHD, bn)

        for c in range(NBUF - AHD):
            step(c, c, True, False)

        def body(jj, carry):
            c0 = NBUF * jj + (NBUF - AHD)
            for b in range(NBUF):
                step(c0 + b, (NBUF - AHD + b) % NBUF, False, False)
            return carry

        lax.fori_loop(0, (NCH - NBUF) // NBUF, body, 0)
        for c in range(NCH - AHD, NCH):
            step(c, c % NBUF, False, True)
        for c in range(NCH - NBUF, NCH):
            wait_s(c, c % NBUF)

    return k


def kernel(x, training):
    B, T, C = x.shape
    BT = B * T
    xf = x.reshape(BT, C)

    with jax.ensure_compile_time_eval():
        logp = jnp.log(jnp.array(
            [_MOVE_PROB / 2.0, 1.0 - _MOVE_PROB, _MOVE_PROB / 2.0],
            dtype=jnp.float32))
        step = jax.random.categorical(
            jax.random.key(42), logp, shape=(BT,)).astype(jnp.int32) - 1
        iota = jnp.arange(BT, dtype=jnp.int32)
        idx = iota + step
        idx = idx + 2 * (idx < 0).astype(jnp.int32)
        idx = idx - 2 * (idx >= BT).astype(jnp.int32)
    idx = jnp.where(training != 0, idx, iota)
    out = _make_sc_gather(BT, C)(xf, idx)
    return out.reshape(B, T, C)
